# baseline (speedup 1.0000x reference)
import functools

import jax
import jax.numpy as jnp
from jax.experimental import pallas as pl
from jax.experimental.pallas import tpu as pltpu

_BM = 80
_NBUF = 9


def _fused_body(x_ref, adj_ref, wg_ref, bgcn_ref, wsem_ref, bsem_ref,
                qsem_ref, out_ref, h_scr, hrelu_scr, buf_ref, sem_ref,
                *, p_total, mblks, n_rows):
    nblocks = p_total * mblks

    def copy(j, slot):
        p = jax.lax.div(j, mblks)
        m = jax.lax.rem(j, mblks)
        return pltpu.make_async_copy(
            adj_ref.at[p, pl.ds(m * _BM, _BM), :],
            buf_ref.at[slot],
            sem_ref.at[slot])

    for k in range(_NBUF):
        copy(k, k).start()

    xb = x_ref[...]
    for p in range(p_total):
        h_scr[p] = jnp.dot(xb, wg_ref[p].astype(jnp.bfloat16),
                           preferred_element_type=jnp.float32
                           ).astype(jnp.bfloat16)

    def step(j, att_sums):
        refill = j - 1 + _NBUF

        @pl.when(jnp.logical_and(j > 0, refill < nblocks))
        def _():
            copy(refill, jax.lax.rem(j - 1, _NBUF)).start()

        slot = jax.lax.rem(j, _NBUF)
        copy(j, slot).wait()
        p = jax.lax.div(j, mblks)
        a = buf_ref[slot].astype(jnp.bfloat16)
        acc = jnp.dot(a, h_scr[p], preferred_element_type=jnp.float32)
        acc = acc + bgcn_ref[pl.ds(p, 1), :]
        hr = jnp.maximum(acc, 0.0)
        mrow = (j - p * mblks) * _BM

        @pl.when(p < p_total - 1)
        def _():
            hrelu_scr[pl.ds(p, 1), pl.ds(mrow, _BM)] = hr[None]

        @pl.when(p == p_total - 1)
        def _():
            out_ref[0, pl.ds(mrow, _BM)] = hr
        t = jnp.tanh(jnp.dot(hr, wsem_ref[...],
                             preferred_element_type=jnp.float32)
                     + bsem_ref[...])
        s = jnp.sum(t * qsem_ref[...])
        return tuple(att_sums[q] + jnp.where(p == q, s, 0.0)
                     for q in range(p_total))

    att_sums = jax.lax.fori_loop(
        0, nblocks, step, tuple(jnp.float32(0.0) for _ in range(p_total)))

    logits = [a * (1.0 / n_rows) for a in att_sums]
    m = logits[0]
    for p in range(1, p_total):
        m = jnp.maximum(m, logits[p])
    exps = [jnp.exp(l - m) for l in logits]
    denom = exps[0]
    for p in range(1, p_total):
        denom = denom + exps[p]
    out = (exps[p_total - 1] / denom) * out_ref[0]
    for p in range(p_total - 1):
        out = out + (exps[p] / denom) * hrelu_scr[p]
    out_ref[0] = out


def kernel(x, adjs, sparse, W_gcn, b_gcn, W_sem, b_sem, q_sem):
    p_total, n, _ = adjs.shape
    nhid = W_gcn.shape[2]
    mblks = n // _BM

    return pl.pallas_call(
        functools.partial(_fused_body, p_total=p_total, mblks=mblks,
                          n_rows=n),
        in_specs=[
            pl.BlockSpec(memory_space=pltpu.MemorySpace.VMEM),
            pl.BlockSpec(memory_space=pltpu.MemorySpace.HBM),
            pl.BlockSpec(memory_space=pltpu.MemorySpace.VMEM),
            pl.BlockSpec(memory_space=pltpu.MemorySpace.VMEM),
            pl.BlockSpec(memory_space=pltpu.MemorySpace.VMEM),
            pl.BlockSpec(memory_space=pltpu.MemorySpace.VMEM),
            pl.BlockSpec(memory_space=pltpu.MemorySpace.VMEM),
        ],
        out_specs=pl.BlockSpec(memory_space=pltpu.MemorySpace.VMEM),
        out_shape=jax.ShapeDtypeStruct((1, n, nhid), jnp.float32),
        scratch_shapes=[
            pltpu.VMEM((p_total, n, nhid), jnp.bfloat16),
            pltpu.VMEM((p_total - 1, n, nhid), jnp.float32),
            pltpu.VMEM((_NBUF, _BM, n), jnp.float32),
            pltpu.SemaphoreType.DMA((_NBUF,)),
        ],
    )(x.astype(jnp.bfloat16), adjs, W_gcn, b_gcn, W_sem, b_sem, q_sem)


# speedup vs baseline: 1.0524x; 1.0524x over previous
import functools

import jax
import jax.numpy as jnp
from jax.experimental import pallas as pl
from jax.experimental.pallas import tpu as pltpu

_BM = 200
_NBUF = 4


def _fused_body(x_ref, adj_ref, wg_ref, bgcn_ref, wsem_ref, bsem_ref,
                qsem_ref, out_ref, h_scr, hrelu_scr, buf_ref, sem_ref,
                *, p_total, mblks, n_rows):
    nblocks = p_total * mblks

    def copy(j, slot):
        p = jax.lax.div(j, mblks)
        m = jax.lax.rem(j, mblks)
        return pltpu.make_async_copy(
            adj_ref.at[p, pl.ds(m * _BM, _BM), :],
            buf_ref.at[slot],
            sem_ref.at[slot])

    for k in range(_NBUF):
        copy(k, k).start()

    xb = x_ref[...]
    for p in range(p_total):
        h_scr[p] = jnp.dot(xb, wg_ref[p].astype(jnp.bfloat16),
                           preferred_element_type=jnp.float32
                           ).astype(jnp.bfloat16)

    def step(j, att_sums):
        refill = j - 1 + _NBUF

        @pl.when(jnp.logical_and(j > 0, refill < nblocks))
        def _():
            copy(refill, jax.lax.rem(j - 1, _NBUF)).start()

        slot = jax.lax.rem(j, _NBUF)
        copy(j, slot).wait()
        p = jax.lax.div(j, mblks)
        acc = jnp.dot(buf_ref[slot], h_scr[p],
                      preferred_element_type=jnp.float32)
        acc = acc + bgcn_ref[pl.ds(p, 1), :]
        hr = jnp.maximum(acc, 0.0)
        mrow = (j - p * mblks) * _BM

        @pl.when(p < p_total - 1)
        def _():
            hrelu_scr[pl.ds(p, 1), pl.ds(mrow, _BM)] = hr[None]

        @pl.when(p == p_total - 1)
        def _():
            out_ref[0, pl.ds(mrow, _BM)] = hr
        t = jnp.tanh(jnp.dot(hr, wsem_ref[...],
                             preferred_element_type=jnp.float32)
                     + bsem_ref[...])
        s = jnp.sum(t * qsem_ref[...])
        return tuple(att_sums[q] + jnp.where(p == q, s, 0.0)
                     for q in range(p_total))

    att_sums = jax.lax.fori_loop(
        0, nblocks, step, tuple(jnp.float32(0.0) for _ in range(p_total)))

    logits = [a * (1.0 / n_rows) for a in att_sums]
    m = logits[0]
    for p in range(1, p_total):
        m = jnp.maximum(m, logits[p])
    exps = [jnp.exp(l - m) for l in logits]
    denom = exps[0]
    for p in range(1, p_total):
        denom = denom + exps[p]
    out = (exps[p_total - 1] / denom) * out_ref[0]
    for p in range(p_total - 1):
        out = out + (exps[p] / denom) * hrelu_scr[p]
    out_ref[0] = out


def kernel(x, adjs, sparse, W_gcn, b_gcn, W_sem, b_sem, q_sem):
    p_total, n, _ = adjs.shape
    nhid = W_gcn.shape[2]
    mblks = n // _BM

    return pl.pallas_call(
        functools.partial(_fused_body, p_total=p_total, mblks=mblks,
                          n_rows=n),
        in_specs=[
            pl.BlockSpec(memory_space=pltpu.MemorySpace.VMEM),
            pl.BlockSpec(memory_space=pltpu.MemorySpace.HBM),
            pl.BlockSpec(memory_space=pltpu.MemorySpace.VMEM),
            pl.BlockSpec(memory_space=pltpu.MemorySpace.VMEM),
            pl.BlockSpec(memory_space=pltpu.MemorySpace.VMEM),
            pl.BlockSpec(memory_space=pltpu.MemorySpace.VMEM),
            pl.BlockSpec(memory_space=pltpu.MemorySpace.VMEM),
        ],
        out_specs=pl.BlockSpec(memory_space=pltpu.MemorySpace.VMEM),
        out_shape=jax.ShapeDtypeStruct((1, n, nhid), jnp.float32),
        scratch_shapes=[
            pltpu.VMEM((p_total, n, nhid), jnp.bfloat16),
            pltpu.VMEM((p_total - 1, n, nhid), jnp.float32),
            pltpu.VMEM((_NBUF, _BM, n), jnp.float32),
            pltpu.SemaphoreType.DMA((_NBUF,)),
        ],
    )(x.astype(jnp.bfloat16), adjs, W_gcn, b_gcn, W_sem, b_sem, q_sem)
